# Initial kernel scaffold; baseline (speedup 1.0000x reference)
import jax
import jax.numpy as jnp
from jax.experimental import pallas as pl


def kernel(x, batch, W, b):
    raise NotImplementedError("write your pallas kernel here")


# speedup vs baseline: 5.2403x; 5.2403x over previous
import functools

import jax
import jax.numpy as jnp
from jax import lax
from jax.experimental import pallas as pl
from jax.experimental.pallas import tpu as pltpu
from jax.experimental.pallas import tpu_sc as plsc

N_NODES = 100000
D = 128
G = 512
ACC_ROWS = 528
CNT_W = 16
AW = D + CNT_W
BLK = 128
NB_FULL = N_NODES // BLK
TAIL = N_NODES - NB_FULL * BLK
NC = 2
NS = 16
NW = NC * NS
QB, RB = divmod(NB_FULL, NW)
MAXB = QB + 1


def _sc_segment_sums(x, batch, zacc, ones):
    mesh = plsc.VectorSubcoreMesh(core_axis_name="c", subcore_axis_name="s")

    @functools.partial(
        pl.kernel,
        mesh=mesh,
        compiler_params=pltpu.CompilerParams(use_tc_tiling_on_sc=False),
        out_type=jax.ShapeDtypeStruct((NC, ACC_ROWS, AW), jnp.float32),
        scratch_types=[
            pltpu.VMEM((BLK, AW), jnp.float32),
            pltpu.VMEM((BLK,), jnp.int32),
            pltpu.VMEM((TAIL,), jnp.int32),
            pltpu.VMEM_SHARED((ACC_ROWS, AW), jnp.float32),
        ],
    )
    def k(x_hbm, b_hbm, zacc_hbm, ones_hbm, part_hbm, xbuf, idxb, idxt, acc):
        cid = lax.axis_index("c")
        sid = lax.axis_index("s")
        wid = sid * NC + cid

        @pl.when(sid == 0)
        def _():
            pltpu.sync_copy(zacc_hbm, acc)

        pltpu.sync_copy(ones_hbm, xbuf.at[:, pl.ds(D, CNT_W)])
        plsc.subcore_barrier()

        start = wid * QB + jnp.minimum(wid, RB)
        nblk = QB + jnp.where(wid < RB, 1, 0)

        def body(i, carry):
            @pl.when(i < nblk)
            def _():
                blk = start + i
                pltpu.sync_copy(x_hbm.at[pl.ds(blk * BLK, BLK)],
                                xbuf.at[:, pl.ds(0, D)])
                pltpu.sync_copy(b_hbm.at[pl.ds(blk * BLK, BLK)], idxb)
                pltpu.sync_copy(xbuf, acc.at[idxb], add=True)
            return carry

        lax.fori_loop(0, MAXB, body, 0)

        @pl.when(wid == NW - 1)
        def _():
            pltpu.sync_copy(x_hbm.at[pl.ds(NB_FULL * BLK, TAIL)],
                            xbuf.at[pl.ds(0, TAIL), pl.ds(0, D)])
            pltpu.sync_copy(b_hbm.at[pl.ds(NB_FULL * BLK, TAIL)], idxt)
            pltpu.sync_copy(xbuf.at[pl.ds(0, TAIL)], acc.at[idxt], add=True)

        plsc.subcore_barrier()

        @pl.when(sid == 0)
        def _():
            pltpu.sync_copy(acc, part_hbm.at[cid])

    return k(x, batch, zacc, ones)


def _tc_head(part, W, b2):
    t = W.shape[1]

    def body(p_ref, w_ref, b_ref, o_ref):
        tot = p_ref[0] + p_ref[1]
        sums = tot[:G, :D]
        counts = tot[:G, D:D + 1]
        rep = sums / jnp.maximum(counts, 1.0)
        o_ref[...] = (
            jnp.dot(rep, w_ref[...], preferred_element_type=jnp.float32)
            + b_ref[...]
        )

    return pl.pallas_call(
        body,
        out_shape=jax.ShapeDtypeStruct((G, t), jnp.float32),
    )(part, W, b2)


def kernel(x, batch, W, b):
    zacc = jnp.zeros((ACC_ROWS, AW), jnp.float32)
    ones = jnp.ones((BLK, CNT_W), jnp.float32)
    part = _sc_segment_sums(x, batch.astype(jnp.int32), zacc, ones)
    return _tc_head(part, W, b.reshape(1, -1))


# speedup vs baseline: 6.6197x; 1.2632x over previous
import functools

import jax
import jax.numpy as jnp
from jax import lax
from jax.experimental import pallas as pl
from jax.experimental.pallas import tpu as pltpu
from jax.experimental.pallas import tpu_sc as plsc

N_NODES = 100000
D = 128
G = 512
ACC_ROWS = 528
CNT_W = 16
AW = D + CNT_W
BLK = 128
NB_FULL = N_NODES // BLK
TAIL = N_NODES - NB_FULL * BLK
NC = 2
NS = 16
NW = NC * NS
QB, RB = divmod(NB_FULL, NW)
MAXB = QB + 1


ROWS_PER_TILE = ACC_ROWS // NS
IDX_ROWS = NW * QB + RB + 1


def _sc_segment_sums(x, b2d, batch, zacc, ones):
    mesh = plsc.VectorSubcoreMesh(core_axis_name="c", subcore_axis_name="s")

    @functools.partial(
        pl.kernel,
        mesh=mesh,
        compiler_params=pltpu.CompilerParams(use_tc_tiling_on_sc=False),
        out_type=jax.ShapeDtypeStruct((NC, ACC_ROWS, AW), jnp.float32),
        scratch_types=[
            pltpu.VMEM((BLK, AW), jnp.float32),
            pltpu.VMEM((BLK, AW), jnp.float32),
            pltpu.VMEM((MAXB, BLK), jnp.int32),
            pltpu.VMEM((TAIL,), jnp.int32),
            pltpu.VMEM_SHARED((ACC_ROWS, AW), jnp.float32),
            pltpu.SemaphoreType.DMA,
            pltpu.SemaphoreType.DMA,
        ],
    )
    def k(x_hbm, b2d_hbm, b_hbm, zacc_hbm, ones_hbm, part_hbm,
          xbuf0, xbuf1, idxall, idxt, acc, gsem0, gsem1):
        cid = lax.axis_index("c")
        sid = lax.axis_index("s")
        wid = sid * NC + cid
        bufs = ((xbuf0, gsem0), (xbuf1, gsem1))

        pltpu.sync_copy(zacc_hbm.at[pl.ds(sid * ROWS_PER_TILE, ROWS_PER_TILE)],
                        acc.at[pl.ds(sid * ROWS_PER_TILE, ROWS_PER_TILE)])

        pltpu.sync_copy(ones_hbm, xbuf0.at[:, pl.ds(D, CNT_W)])
        pltpu.sync_copy(ones_hbm, xbuf1.at[:, pl.ds(D, CNT_W)])

        start = wid * QB + jnp.minimum(wid, RB)
        nblk = QB + jnp.where(wid < RB, 1, 0)

        pltpu.sync_copy(b2d_hbm.at[pl.ds(start, MAXB)], idxall)
        plsc.subcore_barrier()

        def gather_x(i, buf, sem):
            return pltpu.make_async_copy(
                x_hbm.at[pl.ds((start + i) * BLK, BLK)],
                buf.at[:, pl.ds(0, D)], sem)

        gather_x(0, xbuf0, gsem0).start()

        def body(i2, carry):
            for b in range(2):
                i = i2 * 2 + b
                buf, sem = bufs[b]
                nbuf, nsem = bufs[1 - b]

                @pl.when(i < nblk)
                def _():
                    @pl.when(i + 1 < nblk)
                    def _():
                        gather_x(i + 1, nbuf, nsem).start()
                    gather_x(i, buf, sem).wait()
                    pltpu.sync_copy(buf, acc.at[idxall.at[i]], add=True)
            return carry

        lax.fori_loop(0, (MAXB + 1) // 2, body, 0)

        @pl.when(wid == NW - 1)
        def _():
            pltpu.sync_copy(x_hbm.at[pl.ds(NB_FULL * BLK, TAIL)],
                            xbuf0.at[pl.ds(0, TAIL), pl.ds(0, D)])
            pltpu.sync_copy(b_hbm.at[pl.ds(NB_FULL * BLK, TAIL)], idxt)
            pltpu.sync_copy(xbuf0.at[pl.ds(0, TAIL)], acc.at[idxt], add=True)

        plsc.subcore_barrier()

        pltpu.sync_copy(acc.at[pl.ds(sid * ROWS_PER_TILE, ROWS_PER_TILE)],
                        part_hbm.at[cid, pl.ds(sid * ROWS_PER_TILE,
                                               ROWS_PER_TILE)])

    return k(x, b2d, batch, zacc, ones)


def _tc_head(part, W, b2):
    t = W.shape[1]

    def body(p_ref, w_ref, b_ref, o_ref):
        tot = p_ref[0] + p_ref[1]
        sums = tot[:G, :D]
        counts = tot[:G, D:D + 1]
        rep = sums / jnp.maximum(counts, 1.0)
        o_ref[...] = (
            jnp.dot(rep, w_ref[...], preferred_element_type=jnp.float32)
            + b_ref[...]
        )

    return pl.pallas_call(
        body,
        out_shape=jax.ShapeDtypeStruct((G, t), jnp.float32),
    )(part, W, b2)


def kernel(x, batch, W, b):
    batch = batch.astype(jnp.int32)
    b2d = jnp.pad(batch, (0, IDX_ROWS * BLK - N_NODES)).reshape(IDX_ROWS, BLK)
    zacc = jnp.zeros((ACC_ROWS, AW), jnp.float32)
    ones = jnp.ones((BLK, CNT_W), jnp.float32)
    part = _sc_segment_sums(x, b2d, batch, zacc, ones)
    return _tc_head(part, W, b.reshape(1, -1))


# speedup vs baseline: 6.8470x; 1.0343x over previous
import functools

import jax
import jax.numpy as jnp
from jax import lax
from jax.experimental import pallas as pl
from jax.experimental.pallas import tpu as pltpu
from jax.experimental.pallas import tpu_sc as plsc

N_NODES = 100000
D = 128
G = 512
ACC_ROWS = 528
CNT_W = 16
AW = D + CNT_W
BLK = 128
NB_FULL = N_NODES // BLK
TAIL = N_NODES - NB_FULL * BLK
NC = 2
NS = 16
NW = NC * NS
QB, RB = divmod(NB_FULL, NW)
MAXB = QB + 1


ROWS_PER_TILE = ACC_ROWS // NS
LANES = 16


def _sc_segment_sums(x, batch):
    mesh = plsc.VectorSubcoreMesh(core_axis_name="c", subcore_axis_name="s")

    @functools.partial(
        pl.kernel,
        mesh=mesh,
        compiler_params=pltpu.CompilerParams(use_tc_tiling_on_sc=False),
        out_type=jax.ShapeDtypeStruct((NC, ACC_ROWS, AW), jnp.float32),
        scratch_types=[
            pltpu.VMEM((BLK, AW), jnp.float32),
            pltpu.VMEM((BLK, AW), jnp.float32),
            pltpu.VMEM((MAXB, BLK), jnp.int32),
            pltpu.VMEM((QB * BLK,), jnp.int32),
            pltpu.VMEM((BLK,), jnp.int32),
            pltpu.VMEM((TAIL,), jnp.int32),
            pltpu.VMEM_SHARED((ACC_ROWS, AW), jnp.float32),
            pltpu.SemaphoreType.DMA,
            pltpu.SemaphoreType.DMA,
        ],
    )
    def k(x_hbm, b_hbm, part_hbm,
          xbuf0, xbuf1, idxall, idx1d, idxx, idxt, acc, gsem0, gsem1):
        cid = lax.axis_index("c")
        sid = lax.axis_index("s")
        wid = sid * NC + cid
        bufs = ((xbuf0, gsem0), (xbuf1, gsem1))

        start = wid * QB + jnp.minimum(wid, RB)
        nblk = QB + jnp.where(wid < RB, 1, 0)

        pltpu.sync_copy(b_hbm.at[pl.ds(start * BLK, QB * BLK)], idx1d)

        @pl.when(wid < RB)
        def _():
            pltpu.sync_copy(b_hbm.at[pl.ds((start + QB) * BLK, BLK)], idxx)

        zeros16 = jnp.zeros((LANES,), jnp.float32)
        for r in range(ROWS_PER_TILE):
            for c in range(AW // LANES):
                xbuf0[r, pl.ds(c * LANES, LANES)] = zeros16
        pltpu.sync_copy(xbuf0.at[pl.ds(0, ROWS_PER_TILE)],
                        acc.at[pl.ds(sid * ROWS_PER_TILE, ROWS_PER_TILE)])

        ones16 = jnp.ones((LANES,), jnp.float32)
        for r in range(BLK):
            xbuf0[r, pl.ds(D, LANES)] = ones16
            xbuf1[r, pl.ds(D, LANES)] = ones16

        for i in range(QB):
            for g in range(BLK // LANES):
                idxall[i, pl.ds(g * LANES, LANES)] = (
                    idx1d[pl.ds(i * BLK + g * LANES, LANES)])

        @pl.when(wid < RB)
        def _():
            for g in range(BLK // LANES):
                idxall[QB, pl.ds(g * LANES, LANES)] = (
                    idxx[pl.ds(g * LANES, LANES)])

        plsc.subcore_barrier()

        def gather_x(i, buf, sem):
            return pltpu.make_async_copy(
                x_hbm.at[pl.ds((start + i) * BLK, BLK)],
                buf.at[:, pl.ds(0, D)], sem)

        gather_x(0, xbuf0, gsem0).start()

        def body(i2, carry):
            for b in range(2):
                i = i2 * 2 + b
                buf, gsem = bufs[b]
                nbuf, ngsem = bufs[1 - b]

                @pl.when(i < nblk)
                def _():
                    @pl.when(i + 1 < nblk)
                    def _():
                        gather_x(i + 1, nbuf, ngsem).start()
                    gather_x(i, buf, gsem).wait()
                    pltpu.sync_copy(buf, acc.at[idxall.at[i]], add=True)
            return carry

        lax.fori_loop(0, (MAXB + 1) // 2, body, 0)

        @pl.when(wid == NW - 1)
        def _():
            pltpu.sync_copy(x_hbm.at[pl.ds(NB_FULL * BLK, TAIL)],
                            xbuf0.at[pl.ds(0, TAIL), pl.ds(0, D)])
            pltpu.sync_copy(b_hbm.at[pl.ds(NB_FULL * BLK, TAIL)], idxt)
            pltpu.sync_copy(xbuf0.at[pl.ds(0, TAIL)], acc.at[idxt], add=True)

        plsc.subcore_barrier()

        pltpu.sync_copy(acc.at[pl.ds(sid * ROWS_PER_TILE, ROWS_PER_TILE)],
                        part_hbm.at[cid, pl.ds(sid * ROWS_PER_TILE,
                                               ROWS_PER_TILE)])

    return k(x, batch)


def _tc_head(part, W, b2):
    t = W.shape[1]

    def body(p_ref, w_ref, b_ref, o_ref):
        tot = p_ref[0] + p_ref[1]
        sums = tot[:G, :D]
        counts = tot[:G, D:D + 1]
        rep = sums / jnp.maximum(counts, 1.0)
        o_ref[...] = (
            jnp.dot(rep, w_ref[...], preferred_element_type=jnp.float32)
            + b_ref[...]
        )

    return pl.pallas_call(
        body,
        out_shape=jax.ShapeDtypeStruct((G, t), jnp.float32),
    )(part, W, b2)


def kernel(x, batch, W, b):
    part = _sc_segment_sums(x, batch.astype(jnp.int32))
    return _tc_head(part, W, b.reshape(1, -1))
